# baseline (speedup 1.0000x reference)
import functools

import jax
import jax.numpy as jnp
from jax import lax
from jax.experimental import pallas as pl
from jax.experimental.pallas import tpu as pltpu
from jax.experimental.pallas import tpu_sc as plsc

N = 10000
NP = 10240
D = 128
E_RAW = 320000
E_AUG = E_RAW + N
NT = 32
CH = 128
NCHUNK = 6 * (-(-E_AUG // (NT * CH * 6)))
EW_T = NCHUNK * CH
EA = EW_T * NT
ROWS_T = NP // 16
EPS = 1e-5

_mesh = plsc.VectorSubcoreMesh(core_axis_name="c", subcore_axis_name="s")
_sc_params = pltpu.CompilerParams(needs_layout_passes=False)


def _sc_pre_body(dst_hbm, ew_hbm, ids_hbm, emb_hbm, degp_hbm, draw_hbm,
                 idx_v, ew_v, acc_v, ids_v, erows_v, sem):
    c = lax.axis_index("c")
    s = lax.axis_index("s")
    w = c * 16 + s

    def zbody(i, _):
        acc_v[pl.ds(i * 16, 16)] = jnp.zeros((16,), jnp.float32)
        return 0

    lax.fori_loop(0, NP // 16, zbody, 0)

    base = w * EW_T

    def chunk(g, _):
        e0 = base + g * CH
        pltpu.sync_copy(dst_hbm.at[pl.ds(e0, CH)], idx_v)
        pltpu.sync_copy(ew_hbm.at[pl.ds(e0, CH)], ew_v)
        for j in range(CH // 16):
            dv = idx_v[pl.ds(j * 16, 16)]
            wv = ew_v[pl.ds(j * 16, 16)]
            plsc.addupdate_scatter(acc_v, [dv], wv)
        return 0

    lax.fori_loop(0, NCHUNK, chunk, 0)
    pltpu.sync_copy(acc_v, degp_hbm.at[w])

    nb = w * (NP // NT)

    def echunk(k, _):
        pltpu.sync_copy(ids_hbm.at[pl.ds(nb + k * 80, 80)], ids_v)
        pltpu.async_copy(emb_hbm.at[ids_v], erows_v, sem).wait()
        pltpu.sync_copy(erows_v, draw_hbm.at[pl.ds(nb + k * 80, 80)])
        return 0

    lax.fori_loop(0, 4, echunk, 0)


_sc_pre = pl.kernel(
    _sc_pre_body,
    out_type=(jax.ShapeDtypeStruct((NT, NP), jnp.float32),
              jax.ShapeDtypeStruct((NP, D), jnp.float32)),
    mesh=_mesh,
    compiler_params=_sc_params,
    scratch_types=[
        pltpu.VMEM((CH,), jnp.int32),
        pltpu.VMEM((CH,), jnp.float32),
        pltpu.VMEM((NP,), jnp.float32),
        pltpu.VMEM((80,), jnp.int32),
        pltpu.VMEM((80, D), jnp.float32),
        pltpu.SemaphoreType.DMA,
    ],
)


def _sc_conv_body(xw_hbm, edata_hbm, dis_hbm,
                  out_hbm, dis_v, cf_v, ib0, ib1, ib2, rows_a, rows_b,
                  acc_sh, is0, is1, is2, gs0, gs1, ss0, ss1):
    c = lax.axis_index("c")
    s = lax.axis_index("s")
    w = c * 16 + s

    w = (1 - c) * 16 + s
    ibs = (ib0, ib1, ib2)
    isems = (is0, is1, is2)
    rvs = (rows_a, rows_b)
    gsems = (gs0, gs1)
    ssems = (ss0, ss1)

    pltpu.sync_copy(dis_hbm, dis_v)

    def zrow(e, _):
        for jj in range(D // 16):
            rows_a[e, pl.ds(jj * 16, 16)] = jnp.zeros((16,), jnp.float32)
        return 0

    lax.fori_loop(0, CH, zrow, 0)
    for k in range(ROWS_T // CH):
        pltpu.sync_copy(rows_a, acc_sh.at[pl.ds(s * ROWS_T + k * CH, CH)])

    pltpu.sync_copy(edata_hbm.at[w, 0], ib0)
    pltpu.async_copy(xw_hbm.at[ib0.at[0]], rows_a, gs0)
    pltpu.async_copy(edata_hbm.at[w, 1], ib1, is1)

    plsc.subcore_barrier()

    def chunk(g, u):
        p, q = u % 2, 1 - (u % 2)
        r = u % 3
        ib, rv = ibs[r], rvs[p]

        @pl.when(g + 2 < NCHUNK)
        def _():
            pltpu.async_copy(edata_hbm.at[w, g + 2], ibs[(u + 2) % 3],
                             isems[(u + 2) % 3])

        @pl.when(g + 1 < NCHUNK)
        def _():
            pltpu.make_async_copy(edata_hbm.at[w, g + 1],
                                  ibs[(u + 1) % 3],
                                  isems[(u + 1) % 3]).wait()
            pltpu.async_copy(xw_hbm.at[ibs[(u + 1) % 3].at[0]], rvs[q],
                             gsems[q])

        pltpu.make_async_copy(xw_hbm.at[ib.at[0]], rv, gsems[p]).wait()
        for j in range(CH // 16):
            sj = ib[0, pl.ds(j * 16, 16)]
            dj = ib[1, pl.ds(j * 16, 16)]
            wj = plsc.bitcast(ib[2, pl.ds(j * 16, 16)], jnp.float32)
            cf_v[pl.ds(j * 16, 16)] = (
                plsc.load_gather(dis_v, [sj]) * wj
                * plsc.load_gather(dis_v, [dj]))

        def scale(e4, _, rv=rv):
            for t in range(4):
                e = e4 * 4 + t
                fv = plsc.load_gather(cf_v, [jnp.broadcast_to(e, (16,))])
                for jj in range(D // 16):
                    rv[e, pl.ds(jj * 16, 16)] = (
                        rv[e, pl.ds(jj * 16, 16)] * fv)
            return 0

        lax.fori_loop(0, CH // 4, scale, 0)
        pltpu.sync_copy(rv, acc_sh.at[ib.at[1]], add=True)

    def six(t, _):
        for u in range(6):
            chunk(t * 6 + u, u)
        return 0

    lax.fori_loop(0, NCHUNK // 6, six, 0)
    plsc.subcore_barrier()
    pltpu.sync_copy(acc_sh.at[pl.ds(s * ROWS_T, ROWS_T)],
                    out_hbm.at[c, pl.ds(s * ROWS_T, ROWS_T)])


_sc_conv = pl.kernel(
    _sc_conv_body,
    out_type=jax.ShapeDtypeStruct((2, NP, D), jnp.float32),
    mesh=_mesh,
    compiler_params=_sc_params,
    scratch_types=[
        pltpu.VMEM((NP,), jnp.float32),
        pltpu.VMEM((CH,), jnp.float32),
        pltpu.VMEM((3, CH), jnp.int32),
        pltpu.VMEM((3, CH), jnp.int32),
        pltpu.VMEM((3, CH), jnp.int32),
        pltpu.VMEM((CH, D), jnp.float32),
        pltpu.VMEM((CH, D), jnp.float32),
        pltpu.VMEM_SHARED((NP, D), jnp.float32),
        pltpu.SemaphoreType.DMA,
        pltpu.SemaphoreType.DMA,
        pltpu.SemaphoreType.DMA,
        pltpu.SemaphoreType.DMA,
        pltpu.SemaphoreType.DMA,
        pltpu.SemaphoreType.DMA,
        pltpu.SemaphoreType.DMA,
    ],
)


def _tc1_body(degp_ref, x_ref, W1_ref, dis_ref, xw1_ref):
    deg = jnp.sum(degp_ref[...], axis=0, keepdims=True)
    dis_ref[...] = jnp.where(deg > 0, jax.lax.rsqrt(deg), 0.0)
    xw1_ref[...] = jnp.dot(x_ref[...], W1_ref[...],
                           preferred_element_type=jnp.float32)


def _tc2_body(part_ref, b1_ref, W2_ref, xw2_ref):
    h = part_ref[0, 0:N, :] + part_ref[1, 0:N, :] + b1_ref[...]
    h = jax.nn.relu(h)
    xw2_ref[...] = jnp.dot(h, W2_ref[...],
                           preferred_element_type=jnp.float32)


def _bn_relu(v, g, b):
    mu = jnp.sum(v, axis=0, keepdims=True) * (1.0 / N)
    var = jnp.sum((v - mu) ** 2, axis=0, keepdims=True) * (1.0 / N)
    return jax.nn.relu(g * (v - mu) * jax.lax.rsqrt(var + EPS) + b)


def _tc3_body(part_ref, b2_ref, draw_ref, edges_ref, W0_ref, b0_ref,
              bn_g_ref, bn_b_ref, be_g_ref, be_b_ref, bd_g_ref, bd_b_ref,
              Wf_ref, bf_ref, out_ref):
    h = part_ref[0, 0:N, :] + part_ref[1, 0:N, :] + b2_ref[...]
    h = _bn_relu(h, bn_g_ref[...], bn_b_ref[...])
    e = edges_ref[...] * W0_ref[...] + b0_ref[...]
    e = _bn_relu(e, be_g_ref[...], be_b_ref[...])
    d = _bn_relu(draw_ref[0:N, :], bd_g_ref[...], bd_b_ref[...])
    Wf = Wf_ref[...]
    acc = jnp.dot(h, Wf[0:D], preferred_element_type=jnp.float32)
    acc += jnp.dot(e, Wf[D:2 * D], preferred_element_type=jnp.float32)
    acc += jnp.dot(d, Wf[2 * D:3 * D], preferred_element_type=jnp.float32)
    out_ref[...] = acc + bf_ref[...]


def kernel(x, edge_index, edge_weight, edges, degree, W1, b1, W2, b2, bn_g,
           bn_b, be_g, be_b, bd_g, bd_b, W0, b0, emb, Wf, bf):
    src, dst = edge_index[0], edge_index[1]
    idt = src.dtype
    pad = EA - E_AUG
    loop = jnp.arange(N, dtype=idt)
    pad_dst = (N + jnp.arange(pad, dtype=idt) % (NP - N)).astype(idt)
    srcA = jnp.concatenate([src, loop, jnp.zeros((pad,), idt)])
    dstA = jnp.concatenate([dst, loop, pad_dst])
    ewA = jnp.concatenate([edge_weight, jnp.ones((N,), jnp.float32),
                           jnp.zeros((pad,), jnp.float32)])
    ids_p = jnp.concatenate([degree, jnp.zeros((NP - N,), degree.dtype)])
    edata = jnp.stack(
        [srcA.reshape(NT, NCHUNK, CH),
         dstA.reshape(NT, NCHUNK, CH),
         jax.lax.bitcast_convert_type(ewA, jnp.int32).reshape(NT, NCHUNK, CH)],
        axis=2)

    degp, d_raw = _sc_pre(dstA, ewA, ids_p, emb)

    dis2d, xw1 = pl.pallas_call(
        _tc1_body,
        out_shape=(jax.ShapeDtypeStruct((1, NP), jnp.float32),
                   jax.ShapeDtypeStruct((N, D), jnp.float32)),
    )(degp, x, W1)
    dis = dis2d.reshape(NP)

    part1 = _sc_conv(xw1, edata, dis)

    xw2 = pl.pallas_call(
        _tc2_body,
        out_shape=jax.ShapeDtypeStruct((N, D), jnp.float32),
    )(part1, b1.reshape(1, D), W2)

    part2 = _sc_conv(xw2, edata, dis)

    return pl.pallas_call(
        _tc3_body,
        out_shape=jax.ShapeDtypeStruct((N, D), jnp.float32),
    )(part2, b2.reshape(1, D), d_raw, edges, W0, b0.reshape(1, D),
      bn_g.reshape(1, D), bn_b.reshape(1, D), be_g.reshape(1, D),
      be_b.reshape(1, D), bd_g.reshape(1, D), bd_b.reshape(1, D),
      Wf, bf.reshape(1, D))


# speedup vs baseline: 1.8394x; 1.8394x over previous
import functools

import jax
import jax.numpy as jnp
from jax import lax
from jax.experimental import pallas as pl
from jax.experimental.pallas import tpu as pltpu
from jax.experimental.pallas import tpu_sc as plsc

N = 10000
NP = 10240
D = 128
E_RAW = 320000
E_AUG = E_RAW + N
NT = 32
CH = 128
NCHUNK = -(-E_AUG // (NT * CH))
EW_T = NCHUNK * CH
EA = EW_T * NT
ROWS_T = NP // 16
EPS = 1e-5

_mesh = plsc.VectorSubcoreMesh(core_axis_name="c", subcore_axis_name="s")
_sc_params = pltpu.CompilerParams(needs_layout_passes=False)


def _sc_pre_body(dst_hbm, ew_hbm, ids_hbm, emb_hbm, degp_hbm, draw_hbm,
                 idx_v, ew_v, acc_v, ids_v, erows_v, sem):
    c = lax.axis_index("c")
    s = lax.axis_index("s")
    w = c * 16 + s

    def zbody(i, _):
        acc_v[pl.ds(i * 16, 16)] = jnp.zeros((16,), jnp.float32)
        return 0

    lax.fori_loop(0, NP // 16, zbody, 0)

    base = w * EW_T

    def chunk(g, _):
        e0 = base + g * CH
        pltpu.sync_copy(dst_hbm.at[pl.ds(e0, CH)], idx_v)
        pltpu.sync_copy(ew_hbm.at[pl.ds(e0, CH)], ew_v)
        for j in range(CH // 16):
            dv = idx_v[pl.ds(j * 16, 16)]
            wv = ew_v[pl.ds(j * 16, 16)]
            plsc.addupdate_scatter(acc_v, [dv], wv)
        return 0

    lax.fori_loop(0, NCHUNK, chunk, 0)
    pltpu.sync_copy(acc_v, degp_hbm.at[w])

    nb = w * (NP // NT)

    def echunk(k, _):
        pltpu.sync_copy(ids_hbm.at[pl.ds(nb + k * 80, 80)], ids_v)
        pltpu.async_copy(emb_hbm.at[ids_v], erows_v, sem).wait()
        pltpu.sync_copy(erows_v, draw_hbm.at[pl.ds(nb + k * 80, 80)])
        return 0

    lax.fori_loop(0, 4, echunk, 0)


_sc_pre = pl.kernel(
    _sc_pre_body,
    out_type=(jax.ShapeDtypeStruct((NT, NP), jnp.float32),
              jax.ShapeDtypeStruct((NP, D), jnp.float32)),
    mesh=_mesh,
    compiler_params=_sc_params,
    scratch_types=[
        pltpu.VMEM((CH,), jnp.int32),
        pltpu.VMEM((CH,), jnp.float32),
        pltpu.VMEM((NP,), jnp.float32),
        pltpu.VMEM((80,), jnp.int32),
        pltpu.VMEM((80, D), jnp.float32),
        pltpu.SemaphoreType.DMA,
    ],
)


def _sc_conv_body(xw_hbm, edata_hbm, dis_hbm,
                  out_hbm, dis_v, cf_v, ib, rows_v, acc_sh, sem):
    c = lax.axis_index("c")
    s = lax.axis_index("s")
    w = c * 16 + s

    pltpu.sync_copy(dis_hbm, dis_v)

    def zrow(e, _):
        for jj in range(D // 16):
            rows_v[e, pl.ds(jj * 16, 16)] = jnp.zeros((16,), jnp.float32)
        return 0

    lax.fori_loop(0, CH, zrow, 0)
    for k in range(ROWS_T // CH):
        pltpu.sync_copy(rows_v, acc_sh.at[pl.ds(s * ROWS_T + k * CH, CH)])
    plsc.subcore_barrier()

    def chunk(g, _):
        pltpu.sync_copy(edata_hbm.at[w, g], ib)
        pltpu.async_copy(xw_hbm.at[ib.at[0]], rows_v, sem).wait()
        for j in range(CH // 16):
            sj = ib[0, pl.ds(j * 16, 16)]
            dj = ib[1, pl.ds(j * 16, 16)]
            wj = plsc.bitcast(ib[2, pl.ds(j * 16, 16)], jnp.float32)
            cf_v[pl.ds(j * 16, 16)] = (
                plsc.load_gather(dis_v, [sj]) * wj
                * plsc.load_gather(dis_v, [dj]))

        def scale(e4, _):
            for t in range(4):
                e = e4 * 4 + t
                fv = plsc.load_gather(cf_v, [jnp.broadcast_to(e, (16,))])
                for jj in range(D // 16):
                    rows_v[e, pl.ds(jj * 16, 16)] = (
                        rows_v[e, pl.ds(jj * 16, 16)] * fv)
            return 0

        lax.fori_loop(0, CH // 4, scale, 0)
        pltpu.sync_copy(rows_v, acc_sh.at[ib.at[1]], add=True)
        return 0

    lax.fori_loop(0, NCHUNK, chunk, 0)
    plsc.subcore_barrier()
    pltpu.sync_copy(acc_sh.at[pl.ds(s * ROWS_T, ROWS_T)],
                    out_hbm.at[c, pl.ds(s * ROWS_T, ROWS_T)])


_sc_conv = pl.kernel(
    _sc_conv_body,
    out_type=jax.ShapeDtypeStruct((2, NP, D), jnp.float32),
    mesh=_mesh,
    compiler_params=_sc_params,
    scratch_types=[
        pltpu.VMEM((NP,), jnp.float32),
        pltpu.VMEM((CH,), jnp.float32),
        pltpu.VMEM((3, CH), jnp.int32),
        pltpu.VMEM((CH, D), jnp.float32),
        pltpu.VMEM_SHARED((NP, D), jnp.float32),
        pltpu.SemaphoreType.DMA,
    ],
)


def _tc1_body(degp_ref, x_ref, W1_ref, dis_ref, xw1_ref):
    deg = jnp.sum(degp_ref[...], axis=0, keepdims=True)
    dis_ref[...] = jnp.where(deg > 0, jax.lax.rsqrt(deg), 0.0)
    xw1_ref[...] = jnp.dot(x_ref[...], W1_ref[...],
                           preferred_element_type=jnp.float32)


def _tc2_body(part_ref, b1_ref, W2_ref, xw2_ref):
    h = part_ref[0, 0:N, :] + part_ref[1, 0:N, :] + b1_ref[...]
    h = jax.nn.relu(h)
    xw2_ref[...] = jnp.dot(h, W2_ref[...],
                           preferred_element_type=jnp.float32)


def _bn_relu(v, g, b):
    mu = jnp.sum(v, axis=0, keepdims=True) * (1.0 / N)
    var = jnp.sum((v - mu) ** 2, axis=0, keepdims=True) * (1.0 / N)
    return jax.nn.relu(g * (v - mu) * jax.lax.rsqrt(var + EPS) + b)


def _tc3_body(part_ref, b2_ref, draw_ref, edges_ref, W0_ref, b0_ref,
              bn_g_ref, bn_b_ref, be_g_ref, be_b_ref, bd_g_ref, bd_b_ref,
              Wf_ref, bf_ref, out_ref):
    h = part_ref[0, 0:N, :] + part_ref[1, 0:N, :] + b2_ref[...]
    h = _bn_relu(h, bn_g_ref[...], bn_b_ref[...])
    e = edges_ref[...] * W0_ref[...] + b0_ref[...]
    e = _bn_relu(e, be_g_ref[...], be_b_ref[...])
    d = _bn_relu(draw_ref[0:N, :], bd_g_ref[...], bd_b_ref[...])
    Wf = Wf_ref[...]
    acc = jnp.dot(h, Wf[0:D], preferred_element_type=jnp.float32)
    acc += jnp.dot(e, Wf[D:2 * D], preferred_element_type=jnp.float32)
    acc += jnp.dot(d, Wf[2 * D:3 * D], preferred_element_type=jnp.float32)
    out_ref[...] = acc + bf_ref[...]


def kernel(x, edge_index, edge_weight, edges, degree, W1, b1, W2, b2, bn_g,
           bn_b, be_g, be_b, bd_g, bd_b, W0, b0, emb, Wf, bf):
    src, dst = edge_index[0], edge_index[1]
    idt = src.dtype
    pad = EA - E_AUG
    loop = jnp.arange(N, dtype=idt)
    pad_dst = (N + jnp.arange(pad, dtype=idt) % (NP - N)).astype(idt)
    srcA = jnp.concatenate([src, loop, jnp.zeros((pad,), idt)])
    dstA = jnp.concatenate([dst, loop, pad_dst])
    ewA = jnp.concatenate([edge_weight, jnp.ones((N,), jnp.float32),
                           jnp.zeros((pad,), jnp.float32)])
    ids_p = jnp.concatenate([degree, jnp.zeros((NP - N,), degree.dtype)])
    edata = jnp.stack(
        [srcA.reshape(NT, NCHUNK, CH),
         dstA.reshape(NT, NCHUNK, CH),
         jax.lax.bitcast_convert_type(ewA, jnp.int32).reshape(NT, NCHUNK, CH)],
        axis=2)

    degp, d_raw = _sc_pre(dstA, ewA, ids_p, emb)

    dis2d, xw1 = pl.pallas_call(
        _tc1_body,
        out_shape=(jax.ShapeDtypeStruct((1, NP), jnp.float32),
                   jax.ShapeDtypeStruct((N, D), jnp.float32)),
    )(degp, x, W1)
    dis = dis2d.reshape(NP)

    part1 = _sc_conv(xw1, edata, dis)

    xw2 = pl.pallas_call(
        _tc2_body,
        out_shape=jax.ShapeDtypeStruct((N, D), jnp.float32),
    )(part1, b1.reshape(1, D), W2)

    part2 = _sc_conv(xw2, edata, dis)

    return pl.pallas_call(
        _tc3_body,
        out_shape=jax.ShapeDtypeStruct((N, D), jnp.float32),
    )(part2, b2.reshape(1, D), d_raw, edges, W0, b0.reshape(1, D),
      bn_g.reshape(1, D), bn_b.reshape(1, D), be_g.reshape(1, D),
      be_b.reshape(1, D), bd_g.reshape(1, D), bd_b.reshape(1, D),
      Wf, bf.reshape(1, D))
